# Initial kernel scaffold; baseline (speedup 1.0000x reference)
import jax
import jax.numpy as jnp
from jax.experimental import pallas as pl


def kernel(x, edge_index, W1, b1, Wg, bg, W2, b2):
    raise NotImplementedError("write your pallas kernel here")


# speedup vs baseline: 32.5957x; 32.5957x over previous
import functools

import jax
import jax.numpy as jnp
from jax import lax
from jax.experimental import pallas as pl
from jax.experimental.pallas import tpu as pltpu, tpu_sc as plsc

N = 10000
NP = 10240
D = 128
E = 320000
NC = 2
NS = 16
NW = NC * NS
K = 64
CB = 16
NB = 10
C = CB * NB
EP = NW * C * K
RPT = NP // NS

_mesh = plsc.VectorSubcoreMesh(core_axis_name="c", subcore_axis_name="s")


@functools.partial(
    pl.kernel,
    out_type=jax.ShapeDtypeStruct((NC, NP), jnp.float32),
    mesh=_mesh,
    scratch_types=[
        pltpu.VMEM((CB, K), jnp.int32),
        pltpu.VMEM((K,), jnp.float32),
        pltpu.VMEM_SHARED((NP,), jnp.float32),
    ],
)
def _sc_degree(dst_hbm, zeros1_hbm, degp_hbm, idx_v, ones_v, deg_sh):
    c = lax.axis_index("c")
    s = lax.axis_index("s")
    wid = c * NS + s
    for t in range(K // 16):
        ones_v[pl.ds(t * 16, 16)] = jnp.ones((16,), jnp.float32)
    rows = pl.ds(pl.multiple_of(s * RPT, 8), RPT)
    pltpu.sync_copy(zeros1_hbm.at[rows], deg_sh.at[rows])
    plsc.subcore_barrier()

    def blk(b, carry):
        pltpu.sync_copy(dst_hbm.at[wid, b], idx_v)

        def body(j, carry2):
            pltpu.sync_copy(ones_v, deg_sh.at[idx_v.at[j]], add=True)
            return carry2

        lax.fori_loop(0, CB, body, 0)
        return carry

    lax.fori_loop(0, NB, blk, 0)
    plsc.subcore_barrier()

    @pl.when(s == 0)
    def _():
        pltpu.sync_copy(deg_sh, degp_hbm.at[c])


@functools.partial(
    pl.kernel,
    out_type=jax.ShapeDtypeStruct((NC, NP, D), jnp.float32),
    mesh=_mesh,
    scratch_types=[
        pltpu.VMEM((CB, K), jnp.int32),
        pltpu.VMEM((CB, K), jnp.int32),
        pltpu.VMEM((K, D), jnp.float32),
        pltpu.VMEM((K, D), jnp.float32),
        pltpu.VMEM_SHARED((NP, D), jnp.float32),
        pltpu.SemaphoreType.DMA,
        pltpu.SemaphoreType.DMA,
    ],
)
def _sc_scatter(u_hbm, src_hbm, dst_hbm, zeros_hbm, p_hbm,
                is_v, id_v, r0, r1, agg_sh, sem0, sem1):
    c = lax.axis_index("c")
    s = lax.axis_index("s")
    wid = c * NS + s
    rows = pl.ds(pl.multiple_of(s * RPT, 8), RPT)

    @pl.when(c == 0)
    def _():
        pltpu.sync_copy(u_hbm.at[rows], agg_sh.at[rows])

    @pl.when(c == 1)
    def _():
        pltpu.sync_copy(zeros_hbm.at[rows], agg_sh.at[rows])

    plsc.subcore_barrier()

    def blk(b, carry):
        pltpu.sync_copy(src_hbm.at[wid, b], is_v)
        pltpu.sync_copy(dst_hbm.at[wid, b], id_v)
        pltpu.async_copy(u_hbm.at[is_v.at[0]], r0, sem0)

        def body(j2, carry2):
            j = j2 * 2
            pltpu.async_copy(u_hbm.at[is_v.at[j + 1]], r1, sem1)
            pltpu.make_async_copy(u_hbm.at[is_v.at[j]], r0, sem0).wait()
            pltpu.sync_copy(r0, agg_sh.at[id_v.at[j]], add=True)

            @pl.when(j + 2 < CB)
            def _():
                pltpu.async_copy(u_hbm.at[is_v.at[j + 2]], r0, sem0)

            pltpu.make_async_copy(u_hbm.at[is_v.at[j + 1]], r1, sem1).wait()
            pltpu.sync_copy(r1, agg_sh.at[id_v.at[j + 1]], add=True)
            return carry2

        lax.fori_loop(0, CB // 2, body, 0)
        return carry

    lax.fori_loop(0, NB, blk, 0)
    plsc.subcore_barrier()
    pltpu.sync_copy(agg_sh.at[rows], p_hbm.at[c, rows])


def _u_body(x_ref, w1_ref, b1_ref, wg_ref, dp_ref, u_ref):
    h = lax.dot_general(x_ref[...], w1_ref[...], (((1,), (1,)), ((), ())),
                        preferred_element_type=jnp.float32)
    h = jnp.maximum(h + b1_ref[...], 0.0)
    h2 = lax.dot_general(h, wg_ref[...], (((1,), (1,)), ((), ())),
                         preferred_element_type=jnp.float32)
    deg = 1.0 + dp_ref[:, 0:1] + dp_ref[:, 1:2]
    u_ref[...] = h2 * lax.rsqrt(deg)


def _f_body(p_ref, dp_ref, bg_ref, w2_ref, b2_ref, o_ref):
    ssum = p_ref[0] + p_ref[1]
    deg = 1.0 + dp_ref[:, 0:1] + dp_ref[:, 1:2]
    gcn = ssum * lax.rsqrt(deg) + bg_ref[...]
    o_ref[...] = lax.dot_general(gcn, w2_ref[...], (((1,), (1,)), ((), ())),
                                 preferred_element_type=jnp.float32) + b2_ref[...]


_R = 1280


def _tc_u(x_pad, W1, b1r, Wg, dpT):
    return pl.pallas_call(
        _u_body,
        grid=(NP // _R,),
        in_specs=[
            pl.BlockSpec((_R, D), lambda i: (i, 0)),
            pl.BlockSpec((D, D), lambda i: (0, 0)),
            pl.BlockSpec((1, D), lambda i: (0, 0)),
            pl.BlockSpec((D, D), lambda i: (0, 0)),
            pl.BlockSpec((_R, 2), lambda i: (i, 0)),
        ],
        out_specs=pl.BlockSpec((_R, D), lambda i: (i, 0)),
        out_shape=jax.ShapeDtypeStruct((NP, D), jnp.float32),
    )(x_pad, W1, b1r, Wg, dpT)


def _tc_final(p, dpT, bgr, W2, b2r):
    return pl.pallas_call(
        _f_body,
        grid=(NP // _R,),
        in_specs=[
            pl.BlockSpec((NC, _R, D), lambda i: (0, i, 0)),
            pl.BlockSpec((_R, 2), lambda i: (i, 0)),
            pl.BlockSpec((1, D), lambda i: (0, 0)),
            pl.BlockSpec((D, D), lambda i: (0, 0)),
            pl.BlockSpec((1, D), lambda i: (0, 0)),
        ],
        out_specs=pl.BlockSpec((_R, D), lambda i: (i, 0)),
        out_shape=jax.ShapeDtypeStruct((NP, D), jnp.float32),
    )(p, dpT, bgr, W2, b2r)


def kernel(x, edge_index, W1, b1, Wg, bg, W2, b2):
    pad = jnp.arange(EP - E, dtype=jnp.int32)
    src = jnp.concatenate([edge_index[0], pad % N]).reshape(NW, NB, CB, K)
    dst = jnp.concatenate([edge_index[1], N + pad % (NP - N)]
                          ).reshape(NW, NB, CB, K)
    x_pad = jnp.pad(x, ((0, NP - N), (0, 0)))
    zeros2 = jnp.zeros((NP, D), jnp.float32)
    zeros1 = jnp.zeros((NP,), jnp.float32)

    degp = _sc_degree(dst, zeros1)
    dpT = degp.T
    u = _tc_u(x_pad, W1, b1.reshape(1, D), Wg, dpT)
    p = _sc_scatter(u, src, dst, zeros2)
    out = _tc_final(p, dpT, bg.reshape(1, D), W2, b2.reshape(1, D))
    return out[:N]


# speedup vs baseline: 36.9993x; 1.1351x over previous
import functools

import jax
import jax.numpy as jnp
from jax import lax
from jax.experimental import pallas as pl
from jax.experimental.pallas import tpu as pltpu, tpu_sc as plsc

N = 10000
NP = 10240
D = 128
E = 320000
NC = 2
NS = 16
NW = NC * NS
K = 128
CB = 8
NB = 10
C = CB * NB
EP = NW * C * K
RPT = NP // NS

_mesh = plsc.VectorSubcoreMesh(core_axis_name="c", subcore_axis_name="s")


@functools.partial(
    pl.kernel,
    out_type=jax.ShapeDtypeStruct((NC, NP), jnp.float32),
    mesh=_mesh,
    scratch_types=[
        pltpu.VMEM((CB, K), jnp.int32),
        pltpu.VMEM((K,), jnp.float32),
        pltpu.VMEM_SHARED((NP,), jnp.float32),
        pltpu.SemaphoreType.DMA,
    ],
)
def _sc_degree(dst_hbm, zeros1_hbm, degp_hbm, idx_v, ones_v, deg_sh, sem_h):
    c = lax.axis_index("c")
    s = lax.axis_index("s")
    wid = c * NS + s
    for t in range(K // 16):
        ones_v[pl.ds(t * 16, 16)] = jnp.ones((16,), jnp.float32)
    rows = pl.ds(pl.multiple_of(s * RPT, 8), RPT)
    pltpu.sync_copy(zeros1_hbm.at[rows], deg_sh.at[rows])
    plsc.subcore_barrier()

    def blk(b, carry):
        pltpu.sync_copy(dst_hbm.at[wid, b], idx_v)
        for j in range(CB):
            pltpu.async_copy(ones_v, deg_sh.at[idx_v.at[j]], sem_h, add=True)
        for j in range(CB):
            pltpu.make_async_copy(ones_v, deg_sh.at[idx_v.at[j]], sem_h).wait()
        return carry

    lax.fori_loop(0, NB, blk, 0)
    plsc.subcore_barrier()

    @pl.when(s == 0)
    def _():
        pltpu.sync_copy(deg_sh, degp_hbm.at[c])


@functools.partial(
    pl.kernel,
    out_type=jax.ShapeDtypeStruct((NC, NP, D), jnp.float32),
    mesh=_mesh,
    scratch_types=[
        pltpu.VMEM((CB, K), jnp.int32),
        pltpu.VMEM((CB, K), jnp.int32),
        pltpu.VMEM((K, D), jnp.float32),
        pltpu.VMEM((K, D), jnp.float32),
        pltpu.VMEM_SHARED((NP, D), jnp.float32),
        pltpu.SemaphoreType.DMA,
        pltpu.SemaphoreType.DMA,
    ],
)
def _sc_scatter(u_hbm, src_hbm, dst_hbm, zeros_hbm, p_hbm,
                is_v, id_v, r0, r1, agg_sh, sem0, sem1):
    c = lax.axis_index("c")
    s = lax.axis_index("s")
    wid = c * NS + s
    rows = pl.ds(pl.multiple_of(s * RPT, 8), RPT)

    @pl.when(c == 0)
    def _():
        pltpu.sync_copy(u_hbm.at[rows], agg_sh.at[rows])

    @pl.when(c == 1)
    def _():
        pltpu.sync_copy(zeros_hbm.at[rows], agg_sh.at[rows])

    plsc.subcore_barrier()

    def blk(b, carry):
        pltpu.sync_copy(src_hbm.at[wid, b], is_v)
        pltpu.sync_copy(dst_hbm.at[wid, b], id_v)
        pltpu.async_copy(u_hbm.at[is_v.at[0]], r0, sem0)

        def body(j2, carry2):
            j = j2 * 2
            pltpu.async_copy(u_hbm.at[is_v.at[j + 1]], r1, sem1)
            pltpu.make_async_copy(u_hbm.at[is_v.at[j]], r0, sem0).wait()
            pltpu.sync_copy(r0, agg_sh.at[id_v.at[j]], add=True)

            @pl.when(j + 2 < CB)
            def _():
                pltpu.async_copy(u_hbm.at[is_v.at[j + 2]], r0, sem0)

            pltpu.make_async_copy(u_hbm.at[is_v.at[j + 1]], r1, sem1).wait()
            pltpu.sync_copy(r1, agg_sh.at[id_v.at[j + 1]], add=True)
            return carry2

        lax.fori_loop(0, CB // 2, body, 0)
        return carry

    lax.fori_loop(0, NB, blk, 0)
    plsc.subcore_barrier()
    pltpu.sync_copy(agg_sh.at[rows], p_hbm.at[c, rows])


def _u_body(x_ref, w1_ref, b1_ref, wg_ref, dp_ref, u_ref):
    h = lax.dot_general(x_ref[...], w1_ref[...], (((1,), (1,)), ((), ())),
                        preferred_element_type=jnp.float32)
    h = jnp.maximum(h + b1_ref[...], 0.0)
    h2 = lax.dot_general(h, wg_ref[...], (((1,), (1,)), ((), ())),
                         preferred_element_type=jnp.float32)
    deg = 1.0 + dp_ref[:, 0:1] + dp_ref[:, 1:2]
    u_ref[...] = h2 * lax.rsqrt(deg)


def _f_body(p_ref, dp_ref, bg_ref, w2_ref, b2_ref, o_ref):
    ssum = p_ref[0] + p_ref[1]
    deg = 1.0 + dp_ref[:, 0:1] + dp_ref[:, 1:2]
    gcn = ssum * lax.rsqrt(deg) + bg_ref[...]
    o_ref[...] = lax.dot_general(gcn, w2_ref[...], (((1,), (1,)), ((), ())),
                                 preferred_element_type=jnp.float32) + b2_ref[...]


_R = 1280


def _tc_u(x_pad, W1, b1r, Wg, dpT):
    return pl.pallas_call(
        _u_body,
        grid=(NP // _R,),
        in_specs=[
            pl.BlockSpec((_R, D), lambda i: (i, 0)),
            pl.BlockSpec((D, D), lambda i: (0, 0)),
            pl.BlockSpec((1, D), lambda i: (0, 0)),
            pl.BlockSpec((D, D), lambda i: (0, 0)),
            pl.BlockSpec((_R, 2), lambda i: (i, 0)),
        ],
        out_specs=pl.BlockSpec((_R, D), lambda i: (i, 0)),
        out_shape=jax.ShapeDtypeStruct((NP, D), jnp.float32),
    )(x_pad, W1, b1r, Wg, dpT)


def _tc_final(p, dpT, bgr, W2, b2r):
    return pl.pallas_call(
        _f_body,
        grid=(NP // _R,),
        in_specs=[
            pl.BlockSpec((NC, _R, D), lambda i: (0, i, 0)),
            pl.BlockSpec((_R, 2), lambda i: (i, 0)),
            pl.BlockSpec((1, D), lambda i: (0, 0)),
            pl.BlockSpec((D, D), lambda i: (0, 0)),
            pl.BlockSpec((1, D), lambda i: (0, 0)),
        ],
        out_specs=pl.BlockSpec((_R, D), lambda i: (i, 0)),
        out_shape=jax.ShapeDtypeStruct((NP, D), jnp.float32),
    )(p, dpT, bgr, W2, b2r)


def kernel(x, edge_index, W1, b1, Wg, bg, W2, b2):
    pad = jnp.arange(EP - E, dtype=jnp.int32)
    src = jnp.concatenate([edge_index[0], pad % N]).reshape(NW, NB, CB, K)
    dst = jnp.concatenate([edge_index[1], N + pad % (NP - N)]
                          ).reshape(NW, NB, CB, K)
    x_pad = jnp.pad(x, ((0, NP - N), (0, 0)))
    zeros2 = jnp.zeros((NP, D), jnp.float32)
    zeros1 = jnp.zeros((NP,), jnp.float32)

    degp = _sc_degree(dst, zeros1)
    dpT = degp.T
    u = _tc_u(x_pad, W1, b1.reshape(1, D), Wg, dpT)
    p = _sc_scatter(u, src, dst, zeros2)
    out = _tc_final(p, dpT, bg.reshape(1, D), W2, b2.reshape(1, D))
    return out[:N]


# speedup vs baseline: 41.2525x; 1.1150x over previous
import functools

import jax
import jax.numpy as jnp
from jax import lax
from jax.experimental import pallas as pl
from jax.experimental.pallas import tpu as pltpu, tpu_sc as plsc

N = 10000
NP = 10240
D = 128
E = 320000
NC = 2
NS = 16
NW = NC * NS
K = 128
CB = 8
NB = 10
C = CB * NB
EP = NW * C * K
RPT = NP // NS

_mesh = plsc.VectorSubcoreMesh(core_axis_name="c", subcore_axis_name="s")


@functools.partial(
    pl.kernel,
    out_type=jax.ShapeDtypeStruct((NC, NP), jnp.float32),
    mesh=_mesh,
    scratch_types=[
        pltpu.VMEM((CB, K), jnp.int32),
        pltpu.VMEM((K,), jnp.float32),
        pltpu.VMEM_SHARED((NP,), jnp.float32),
        pltpu.SemaphoreType.DMA,
    ],
)
def _sc_degree(dst_hbm, zeros1_hbm, degp_hbm, idx_v, ones_v, deg_sh, sem_h):
    c = lax.axis_index("c")
    s = lax.axis_index("s")
    wid = c * NS + s
    for t in range(K // 16):
        ones_v[pl.ds(t * 16, 16)] = jnp.ones((16,), jnp.float32)
    rows = pl.ds(pl.multiple_of(s * RPT, 8), RPT)
    pltpu.sync_copy(zeros1_hbm.at[rows], deg_sh.at[rows])
    plsc.subcore_barrier()

    def blk(b, carry):
        pltpu.sync_copy(dst_hbm.at[wid, b], idx_v)
        for j in range(CB):
            pltpu.async_copy(ones_v, deg_sh.at[idx_v.at[j]], sem_h, add=True)
        for j in range(CB):
            pltpu.make_async_copy(ones_v, deg_sh.at[idx_v.at[j]], sem_h).wait()
        return carry

    lax.fori_loop(0, NB, blk, 0)
    plsc.subcore_barrier()

    @pl.when(s == 0)
    def _():
        pltpu.sync_copy(deg_sh, degp_hbm.at[c])


@functools.partial(
    pl.kernel,
    out_type=jax.ShapeDtypeStruct((NC, NP, D), jnp.float32),
    mesh=_mesh,
    scratch_types=[
        pltpu.VMEM((CB, K), jnp.int32),
        pltpu.VMEM((CB, K), jnp.int32),
        pltpu.VMEM((CB, K), jnp.int32),
        pltpu.VMEM((CB, K), jnp.int32),
        pltpu.VMEM((K, D), jnp.float32),
        pltpu.VMEM((K, D), jnp.float32),
        pltpu.VMEM_SHARED((NP, D), jnp.float32),
        pltpu.SemaphoreType.DMA,
        pltpu.SemaphoreType.DMA,
        pltpu.SemaphoreType.DMA,
    ],
)
def _sc_scatter(u_hbm, src_hbm, dst_hbm, zeros_hbm, p_hbm,
                isA, idA, isB, idB, r0, r1, agg_sh, sem0, sem1, sem_i):
    c = lax.axis_index("c")
    s = lax.axis_index("s")
    wid = c * NS + s
    rows = pl.ds(pl.multiple_of(s * RPT, 8), RPT)

    @pl.when(c == 0)
    def _():
        pltpu.sync_copy(u_hbm.at[rows], agg_sh.at[rows])

    @pl.when(c == 1)
    def _():
        pltpu.sync_copy(zeros_hbm.at[rows], agg_sh.at[rows])

    plsc.subcore_barrier()

    pltpu.sync_copy(src_hbm.at[wid, 0], isA)
    pltpu.sync_copy(dst_hbm.at[wid, 0], idA)
    pltpu.async_copy(u_hbm.at[isA.at[0]], r0, sem0)

    def do_block(b, is_v, id_v, nis, nid):
        @pl.when(b + 1 < NB)
        def _():
            pltpu.async_copy(src_hbm.at[wid, b + 1], nis, sem_i)
            pltpu.async_copy(dst_hbm.at[wid, b + 1], nid, sem_i)

        def body(j2, carry2):
            j = j2 * 2
            pltpu.async_copy(u_hbm.at[is_v.at[j + 1]], r1, sem1)
            pltpu.make_async_copy(u_hbm.at[is_v.at[j]], r0, sem0).wait()
            pltpu.sync_copy(r0, agg_sh.at[id_v.at[j]], add=True)

            @pl.when(j + 2 < CB)
            def _():
                pltpu.async_copy(u_hbm.at[is_v.at[j + 2]], r0, sem0)

            @pl.when(jnp.logical_and(j2 == CB // 2 - 1, b + 1 < NB))
            def _():
                pltpu.make_async_copy(src_hbm.at[wid, b + 1], nis, sem_i).wait()
                pltpu.make_async_copy(dst_hbm.at[wid, b + 1], nid, sem_i).wait()
                pltpu.async_copy(u_hbm.at[nis.at[0]], r0, sem0)

            pltpu.make_async_copy(u_hbm.at[is_v.at[j + 1]], r1, sem1).wait()
            pltpu.sync_copy(r1, agg_sh.at[id_v.at[j + 1]], add=True)
            return carry2

        lax.fori_loop(0, CB // 2, body, 0)

    def blkpair(bb, carry):
        b = bb * 2
        do_block(b, isA, idA, isB, idB)
        do_block(b + 1, isB, idB, isA, idA)
        return carry

    lax.fori_loop(0, NB // 2, blkpair, 0)
    plsc.subcore_barrier()
    pltpu.sync_copy(agg_sh.at[rows], p_hbm.at[c, rows])


def _h2_body(x_ref, w1_ref, b1_ref, wg_ref, h2_ref):
    h = lax.dot_general(x_ref[...], w1_ref[...], (((1,), (1,)), ((), ())),
                        preferred_element_type=jnp.float32)
    h = jnp.maximum(h + b1_ref[...], 0.0)
    h2_ref[...] = lax.dot_general(h, wg_ref[...], (((1,), (1,)), ((), ())),
                                  preferred_element_type=jnp.float32)


def _scale_body(h2_ref, dp_ref, u_ref):
    deg = 1.0 + dp_ref[:, 0:1] + dp_ref[:, 1:2]
    u_ref[...] = h2_ref[...] * lax.rsqrt(deg)


def _f_body(p_ref, dp_ref, bg_ref, w2_ref, b2_ref, o_ref):
    ssum = p_ref[0] + p_ref[1]
    deg = 1.0 + dp_ref[:, 0:1] + dp_ref[:, 1:2]
    gcn = ssum * lax.rsqrt(deg) + bg_ref[...]
    o_ref[...] = lax.dot_general(gcn, w2_ref[...], (((1,), (1,)), ((), ())),
                                 preferred_element_type=jnp.float32) + b2_ref[...]


_R = 1280


def _tc_h2(x_pad, W1, b1r, Wg):
    return pl.pallas_call(
        _h2_body,
        grid=(NP // _R,),
        in_specs=[
            pl.BlockSpec((_R, D), lambda i: (i, 0)),
            pl.BlockSpec((D, D), lambda i: (0, 0)),
            pl.BlockSpec((1, D), lambda i: (0, 0)),
            pl.BlockSpec((D, D), lambda i: (0, 0)),
        ],
        out_specs=pl.BlockSpec((_R, D), lambda i: (i, 0)),
        out_shape=jax.ShapeDtypeStruct((NP, D), jnp.float32),
    )(x_pad, W1, b1r, Wg)


def _tc_scale(h2, dpT):
    return pl.pallas_call(
        _scale_body,
        grid=(NP // _R,),
        in_specs=[
            pl.BlockSpec((_R, D), lambda i: (i, 0)),
            pl.BlockSpec((_R, 2), lambda i: (i, 0)),
        ],
        out_specs=pl.BlockSpec((_R, D), lambda i: (i, 0)),
        out_shape=jax.ShapeDtypeStruct((NP, D), jnp.float32),
    )(h2, dpT)


def _tc_final(p, dpT, bgr, W2, b2r):
    return pl.pallas_call(
        _f_body,
        grid=(NP // _R,),
        in_specs=[
            pl.BlockSpec((NC, _R, D), lambda i: (0, i, 0)),
            pl.BlockSpec((_R, 2), lambda i: (i, 0)),
            pl.BlockSpec((1, D), lambda i: (0, 0)),
            pl.BlockSpec((D, D), lambda i: (0, 0)),
            pl.BlockSpec((1, D), lambda i: (0, 0)),
        ],
        out_specs=pl.BlockSpec((_R, D), lambda i: (i, 0)),
        out_shape=jax.ShapeDtypeStruct((NP, D), jnp.float32),
    )(p, dpT, bgr, W2, b2r)


def kernel(x, edge_index, W1, b1, Wg, bg, W2, b2):
    pad = jnp.arange(EP - E, dtype=jnp.int32)
    src = jnp.concatenate([edge_index[0], pad % N]).reshape(NW, NB, CB, K)
    dst = jnp.concatenate([edge_index[1], N + pad % (NP - N)]
                          ).reshape(NW, NB, CB, K)
    x_pad = jnp.pad(x, ((0, NP - N), (0, 0)))
    zeros2 = jnp.zeros((NP, D), jnp.float32)
    zeros1 = jnp.zeros((NP,), jnp.float32)

    h2 = _tc_h2(x_pad, W1, b1.reshape(1, D), Wg)
    degp = _sc_degree(dst, zeros1)
    dpT = degp.T
    u = _tc_scale(h2, dpT)
    p = _sc_scatter(u, src, dst, zeros2)
    out = _tc_final(p, dpT, bg.reshape(1, D), W2, b2.reshape(1, D))
    return out[:N]


# speedup vs baseline: 42.1583x; 1.0220x over previous
import functools

import jax
import jax.numpy as jnp
from jax import lax
from jax.experimental import pallas as pl
from jax.experimental.pallas import tpu as pltpu, tpu_sc as plsc

N = 10000
NP = 10240
D = 128
E = 320000
NC = 2
NS = 16
NW = NC * NS
K = 128
CB = 8
NB = 10
C = CB * NB
EP = NW * C * K
RPT = NP // NS

_mesh = plsc.VectorSubcoreMesh(core_axis_name="c", subcore_axis_name="s")


@functools.partial(
    pl.kernel,
    out_type=jax.ShapeDtypeStruct((NC, NP), jnp.float32),
    mesh=_mesh,
    scratch_types=[
        pltpu.VMEM((CB, K), jnp.int32),
        pltpu.VMEM((CB, K), jnp.int32),
        pltpu.VMEM((K,), jnp.float32),
        pltpu.VMEM_SHARED((NP,), jnp.float32),
        pltpu.SemaphoreType.DMA,
        pltpu.SemaphoreType.DMA,
        pltpu.SemaphoreType.DMA,
    ],
)
def _sc_degree(dst_hbm, zeros1_hbm, degp_hbm, iA, iB, ones_v, deg_sh,
               semA, semB, sem_i):
    c = lax.axis_index("c")
    s = lax.axis_index("s")
    wid = c * NS + s
    for t in range(K // 16):
        ones_v[pl.ds(t * 16, 16)] = jnp.ones((16,), jnp.float32)
    rows = pl.ds(pl.multiple_of(s * RPT, 8), RPT)
    pltpu.sync_copy(dst_hbm.at[wid, 0], iA)
    pltpu.sync_copy(zeros1_hbm.at[rows], deg_sh.at[rows])
    plsc.subcore_barrier()

    bufs, sems = (iA, iB), (semA, semB)
    for b in range(NB):
        me, sem_me = bufs[b % 2], sems[b % 2]
        nxt, sem_nxt = bufs[(b + 1) % 2], sems[(b + 1) % 2]
        if b > 0:
            pltpu.make_async_copy(dst_hbm.at[wid, b], me, sem_i).wait()
        for j in range(CB):
            pltpu.async_copy(ones_v, deg_sh.at[me.at[j]], sem_me, add=True)
        if b + 1 < NB:
            if b > 0:
                for j in range(CB):
                    pltpu.make_async_copy(ones_v, deg_sh.at[nxt.at[j]],
                                          sem_nxt).wait()
            pltpu.async_copy(dst_hbm.at[wid, b + 1], nxt, sem_i)
    for j in range(CB):
        pltpu.make_async_copy(ones_v, deg_sh.at[iA.at[j]], semA).wait()
    for j in range(CB):
        pltpu.make_async_copy(ones_v, deg_sh.at[iB.at[j]], semB).wait()
    plsc.subcore_barrier()

    @pl.when(s == 0)
    def _():
        pltpu.sync_copy(deg_sh, degp_hbm.at[c])


@functools.partial(
    pl.kernel,
    out_type=jax.ShapeDtypeStruct((NC, NP, D), jnp.float32),
    mesh=_mesh,
    scratch_types=[
        pltpu.VMEM((CB, K), jnp.int32),
        pltpu.VMEM((CB, K), jnp.int32),
        pltpu.VMEM((CB, K), jnp.int32),
        pltpu.VMEM((CB, K), jnp.int32),
        pltpu.VMEM((K, D), jnp.float32),
        pltpu.VMEM((K, D), jnp.float32),
        pltpu.VMEM_SHARED((NP, D), jnp.float32),
        pltpu.SemaphoreType.DMA,
        pltpu.SemaphoreType.DMA,
        pltpu.SemaphoreType.DMA,
    ],
)
def _sc_scatter(u_hbm, src_hbm, dst_hbm, zeros_hbm, p_hbm,
                isA, idA, isB, idB, r0, r1, agg_sh, sem0, sem1, sem_i):
    c = lax.axis_index("c")
    s = lax.axis_index("s")
    wid = c * NS + s
    rows = pl.ds(pl.multiple_of(s * RPT, 8), RPT)

    pltpu.sync_copy(src_hbm.at[wid, 0], isA)
    pltpu.sync_copy(dst_hbm.at[wid, 0], idA)
    pltpu.async_copy(u_hbm.at[isA.at[0]], r0, sem0)

    @pl.when(c == 0)
    def _():
        pltpu.sync_copy(u_hbm.at[rows], agg_sh.at[rows])

    @pl.when(c == 1)
    def _():
        pltpu.sync_copy(zeros_hbm.at[rows], agg_sh.at[rows])

    plsc.subcore_barrier()

    def do_block(b, is_v, id_v, nis, nid):
        @pl.when(b + 1 < NB)
        def _():
            pltpu.async_copy(src_hbm.at[wid, b + 1], nis, sem_i)
            pltpu.async_copy(dst_hbm.at[wid, b + 1], nid, sem_i)

        def body(j2, carry2):
            j = j2 * 2
            pltpu.async_copy(u_hbm.at[is_v.at[j + 1]], r1, sem1)
            pltpu.make_async_copy(u_hbm.at[is_v.at[j]], r0, sem0).wait()
            pltpu.sync_copy(r0, agg_sh.at[id_v.at[j]], add=True)

            @pl.when(j + 2 < CB)
            def _():
                pltpu.async_copy(u_hbm.at[is_v.at[j + 2]], r0, sem0)

            @pl.when(jnp.logical_and(j2 == CB // 2 - 1, b + 1 < NB))
            def _():
                pltpu.make_async_copy(src_hbm.at[wid, b + 1], nis, sem_i).wait()
                pltpu.make_async_copy(dst_hbm.at[wid, b + 1], nid, sem_i).wait()
                pltpu.async_copy(u_hbm.at[nis.at[0]], r0, sem0)

            pltpu.make_async_copy(u_hbm.at[is_v.at[j + 1]], r1, sem1).wait()
            pltpu.sync_copy(r1, agg_sh.at[id_v.at[j + 1]], add=True)
            return carry2

        lax.fori_loop(0, CB // 2, body, 0)

    def blkpair(bb, carry):
        b = bb * 2
        do_block(b, isA, idA, isB, idB)
        do_block(b + 1, isB, idB, isA, idA)
        return carry

    lax.fori_loop(0, NB // 2, blkpair, 0)
    plsc.subcore_barrier()
    pltpu.sync_copy(agg_sh.at[rows], p_hbm.at[c, rows])


def _h2_body(x_ref, w1_ref, b1_ref, wg_ref, h2_ref):
    h = lax.dot_general(x_ref[...], w1_ref[...], (((1,), (1,)), ((), ())),
                        preferred_element_type=jnp.float32)
    h = jnp.maximum(h + b1_ref[...], 0.0)
    h2_ref[...] = lax.dot_general(h, wg_ref[...], (((1,), (1,)), ((), ())),
                                  preferred_element_type=jnp.float32)


def _scale_body(h2_ref, dp_ref, u_ref):
    deg = 1.0 + dp_ref[:, 0:1] + dp_ref[:, 1:2]
    u_ref[...] = h2_ref[...] * lax.rsqrt(deg)


def _f_body(p_ref, dp_ref, bg_ref, w2_ref, b2_ref, o_ref):
    ssum = p_ref[0] + p_ref[1]
    deg = 1.0 + dp_ref[:, 0:1] + dp_ref[:, 1:2]
    gcn = ssum * lax.rsqrt(deg) + bg_ref[...]
    o_ref[...] = lax.dot_general(gcn, w2_ref[...], (((1,), (1,)), ((), ())),
                                 preferred_element_type=jnp.float32) + b2_ref[...]


_R = 1280


def _tc_h2(x_pad, W1, b1r, Wg):
    return pl.pallas_call(
        _h2_body,
        grid=(NP // _R,),
        in_specs=[
            pl.BlockSpec((_R, D), lambda i: (i, 0)),
            pl.BlockSpec((D, D), lambda i: (0, 0)),
            pl.BlockSpec((1, D), lambda i: (0, 0)),
            pl.BlockSpec((D, D), lambda i: (0, 0)),
        ],
        out_specs=pl.BlockSpec((_R, D), lambda i: (i, 0)),
        out_shape=jax.ShapeDtypeStruct((NP, D), jnp.float32),
    )(x_pad, W1, b1r, Wg)


def _tc_scale(h2, dpT):
    return pl.pallas_call(
        _scale_body,
        grid=(NP // _R,),
        in_specs=[
            pl.BlockSpec((_R, D), lambda i: (i, 0)),
            pl.BlockSpec((_R, 2), lambda i: (i, 0)),
        ],
        out_specs=pl.BlockSpec((_R, D), lambda i: (i, 0)),
        out_shape=jax.ShapeDtypeStruct((NP, D), jnp.float32),
    )(h2, dpT)


def _tc_final(p, dpT, bgr, W2, b2r):
    return pl.pallas_call(
        _f_body,
        grid=(NP // _R,),
        in_specs=[
            pl.BlockSpec((NC, _R, D), lambda i: (0, i, 0)),
            pl.BlockSpec((_R, 2), lambda i: (i, 0)),
            pl.BlockSpec((1, D), lambda i: (0, 0)),
            pl.BlockSpec((D, D), lambda i: (0, 0)),
            pl.BlockSpec((1, D), lambda i: (0, 0)),
        ],
        out_specs=pl.BlockSpec((_R, D), lambda i: (i, 0)),
        out_shape=jax.ShapeDtypeStruct((NP, D), jnp.float32),
    )(p, dpT, bgr, W2, b2r)


def kernel(x, edge_index, W1, b1, Wg, bg, W2, b2):
    pad = jnp.arange(EP - E, dtype=jnp.int32)
    src = jnp.concatenate([edge_index[0], pad % N]).reshape(NW, NB, CB, K)
    dst = jnp.concatenate([edge_index[1], N + pad % (NP - N)]
                          ).reshape(NW, NB, CB, K)
    x_pad = jnp.pad(x, ((0, NP - N), (0, 0)))
    zeros2 = jnp.zeros((NP, D), jnp.float32)
    zeros1 = jnp.zeros((NP,), jnp.float32)

    h2 = _tc_h2(x_pad, W1, b1.reshape(1, D), Wg)
    degp = _sc_degree(dst, zeros1)
    dpT = degp.T
    u = _tc_scale(h2, dpT)
    p = _sc_scatter(u, src, dst, zeros2)
    out = _tc_final(p, dpT, bg.reshape(1, D), W2, b2.reshape(1, D))
    return out[:N]


# speedup vs baseline: 42.2255x; 1.0016x over previous
import functools

import jax
import jax.numpy as jnp
from jax import lax
from jax.experimental import pallas as pl
from jax.experimental.pallas import tpu as pltpu, tpu_sc as plsc

N = 10000
NP = 10240
D = 128
E = 320000
NC = 2
NS = 16
NW = NC * NS
K = 128
CB = 8
NB = 10
C = CB * NB
EP = NW * C * K
RPT = NP // NS

_mesh = plsc.VectorSubcoreMesh(core_axis_name="c", subcore_axis_name="s")


@functools.partial(
    pl.kernel,
    out_type=jax.ShapeDtypeStruct((NC, NP), jnp.float32),
    mesh=_mesh,
    scratch_types=[
        pltpu.VMEM((CB, K), jnp.int32),
        pltpu.VMEM((CB, K), jnp.int32),
        pltpu.VMEM((K,), jnp.float32),
        pltpu.VMEM_SHARED((NP,), jnp.float32),
        pltpu.SemaphoreType.DMA,
        pltpu.SemaphoreType.DMA,
        pltpu.SemaphoreType.DMA,
    ],
)
def _sc_degree(dst_hbm, zeros1_hbm, degp_hbm, iA, iB, ones_v, deg_sh,
               semA, semB, sem_i):
    c = lax.axis_index("c")
    s = lax.axis_index("s")
    wid = c * NS + s
    for t in range(K // 16):
        ones_v[pl.ds(t * 16, 16)] = jnp.ones((16,), jnp.float32)
    rows = pl.ds(pl.multiple_of(s * RPT, 8), RPT)
    pltpu.sync_copy(dst_hbm.at[wid, 0], iA)
    pltpu.sync_copy(zeros1_hbm.at[rows], deg_sh.at[rows])
    plsc.subcore_barrier()

    bufs, sems = (iA, iB), (semA, semB)
    for b in range(NB):
        me, sem_me = bufs[b % 2], sems[b % 2]
        nxt, sem_nxt = bufs[(b + 1) % 2], sems[(b + 1) % 2]
        if b > 0:
            pltpu.make_async_copy(dst_hbm.at[wid, b], me, sem_i).wait()
        for j in range(CB):
            pltpu.async_copy(ones_v, deg_sh.at[me.at[j]], sem_me, add=True)
        if b + 1 < NB:
            if b > 0:
                for j in range(CB):
                    pltpu.make_async_copy(ones_v, deg_sh.at[nxt.at[j]],
                                          sem_nxt).wait()
            pltpu.async_copy(dst_hbm.at[wid, b + 1], nxt, sem_i)
    for j in range(CB):
        pltpu.make_async_copy(ones_v, deg_sh.at[iA.at[j]], semA).wait()
    for j in range(CB):
        pltpu.make_async_copy(ones_v, deg_sh.at[iB.at[j]], semB).wait()
    plsc.subcore_barrier()

    @pl.when(s == 0)
    def _():
        pltpu.sync_copy(deg_sh, degp_hbm.at[c])


@functools.partial(
    pl.kernel,
    out_type=jax.ShapeDtypeStruct((NC, NP, D), jnp.float32),
    mesh=_mesh,
    scratch_types=[
        pltpu.VMEM((CB, K), jnp.int32),
        pltpu.VMEM((CB, K), jnp.int32),
        pltpu.VMEM((CB, K), jnp.int32),
        pltpu.VMEM((CB, K), jnp.int32),
        pltpu.VMEM((K, D), jnp.float32),
        pltpu.VMEM((K, D), jnp.float32),
        pltpu.VMEM_SHARED((NP, D), jnp.float32),
        pltpu.SemaphoreType.DMA,
        pltpu.SemaphoreType.DMA,
        pltpu.SemaphoreType.DMA,
    ],
)
def _sc_scatter(u_hbm, src_hbm, dst_hbm, zeros_hbm, p_hbm,
                isA, idA, isB, idB, r0, r1, agg_sh, sem0, sem1, sem_i):
    c = lax.axis_index("c")
    s = lax.axis_index("s")
    wid = c * NS + s
    rows = pl.ds(pl.multiple_of(s * RPT, 8), RPT)

    pltpu.sync_copy(src_hbm.at[wid, 0], isA)
    pltpu.sync_copy(dst_hbm.at[wid, 0], idA)
    pltpu.async_copy(u_hbm.at[isA.at[0]], r0, sem0)

    @pl.when(c == 0)
    def _():
        pltpu.sync_copy(u_hbm.at[rows], agg_sh.at[rows])

    @pl.when(c == 1)
    def _():
        pltpu.sync_copy(zeros_hbm.at[rows], agg_sh.at[rows])

    plsc.subcore_barrier()

    def do_block(b, is_v, id_v, nis, nid):
        @pl.when(b + 1 < NB)
        def _():
            pltpu.async_copy(src_hbm.at[wid, b + 1], nis, sem_i)
            pltpu.async_copy(dst_hbm.at[wid, b + 1], nid, sem_i)

        def body(j2, carry2):
            j = j2 * 2
            pltpu.async_copy(u_hbm.at[is_v.at[j + 1]], r1, sem1)
            pltpu.make_async_copy(u_hbm.at[is_v.at[j]], r0, sem0).wait()
            pltpu.sync_copy(r0, agg_sh.at[id_v.at[j]], add=True)

            @pl.when(j + 2 < CB)
            def _():
                pltpu.async_copy(u_hbm.at[is_v.at[j + 2]], r0, sem0)

            @pl.when(jnp.logical_and(j2 == CB // 2 - 1, b + 1 < NB))
            def _():
                pltpu.make_async_copy(src_hbm.at[wid, b + 1], nis, sem_i).wait()
                pltpu.make_async_copy(dst_hbm.at[wid, b + 1], nid, sem_i).wait()
                pltpu.async_copy(u_hbm.at[nis.at[0]], r0, sem0)

            pltpu.make_async_copy(u_hbm.at[is_v.at[j + 1]], r1, sem1).wait()
            pltpu.sync_copy(r1, agg_sh.at[id_v.at[j + 1]], add=True)
            return carry2

        lax.fori_loop(0, CB // 2, body, 0)

    def blkpair(bb, carry):
        b = bb * 2
        do_block(b, isA, idA, isB, idB)
        do_block(b + 1, isB, idB, isA, idA)
        return carry

    lax.fori_loop(0, NB // 2, blkpair, 0)
    plsc.subcore_barrier()
    pltpu.sync_copy(agg_sh.at[rows], p_hbm.at[c, rows])


def _u_body(x_ref, w1_ref, b1_ref, wg_ref, dp_ref, u_ref):
    h = lax.dot_general(x_ref[...], w1_ref[...], (((1,), (1,)), ((), ())),
                        preferred_element_type=jnp.float32)
    h = jnp.maximum(h + b1_ref[...], 0.0)
    h2 = lax.dot_general(h, wg_ref[...], (((1,), (1,)), ((), ())),
                         preferred_element_type=jnp.float32)
    deg = 1.0 + dp_ref[:, 0:1] + dp_ref[:, 1:2]
    u_ref[...] = h2 * lax.rsqrt(deg)


def _f_body(p_ref, dp_ref, bg_ref, w2_ref, b2_ref, o_ref):
    ssum = p_ref[0] + p_ref[1]
    deg = 1.0 + dp_ref[:, 0:1] + dp_ref[:, 1:2]
    gcn = ssum * lax.rsqrt(deg) + bg_ref[...]
    o_ref[...] = lax.dot_general(gcn, w2_ref[...], (((1,), (1,)), ((), ())),
                                 preferred_element_type=jnp.float32) + b2_ref[...]


_R = 1280


def _tc_u(x_pad, W1, b1r, Wg, dpT):
    return pl.pallas_call(
        _u_body,
        grid=(NP // _R,),
        in_specs=[
            pl.BlockSpec((_R, D), lambda i: (i, 0)),
            pl.BlockSpec((D, D), lambda i: (0, 0)),
            pl.BlockSpec((1, D), lambda i: (0, 0)),
            pl.BlockSpec((D, D), lambda i: (0, 0)),
            pl.BlockSpec((_R, 2), lambda i: (i, 0)),
        ],
        out_specs=pl.BlockSpec((_R, D), lambda i: (i, 0)),
        out_shape=jax.ShapeDtypeStruct((NP, D), jnp.float32),
    )(x_pad, W1, b1r, Wg, dpT)


def _tc_final(p, dpT, bgr, W2, b2r):
    return pl.pallas_call(
        _f_body,
        grid=(NP // _R,),
        in_specs=[
            pl.BlockSpec((NC, _R, D), lambda i: (0, i, 0)),
            pl.BlockSpec((_R, 2), lambda i: (i, 0)),
            pl.BlockSpec((1, D), lambda i: (0, 0)),
            pl.BlockSpec((D, D), lambda i: (0, 0)),
            pl.BlockSpec((1, D), lambda i: (0, 0)),
        ],
        out_specs=pl.BlockSpec((_R, D), lambda i: (i, 0)),
        out_shape=jax.ShapeDtypeStruct((NP, D), jnp.float32),
    )(p, dpT, bgr, W2, b2r)


def kernel(x, edge_index, W1, b1, Wg, bg, W2, b2):
    pad = jnp.arange(EP - E, dtype=jnp.int32)
    src = jnp.concatenate([edge_index[0], pad % N]).reshape(NW, NB, CB, K)
    dst = jnp.concatenate([edge_index[1], N + pad % (NP - N)]
                          ).reshape(NW, NB, CB, K)
    x_pad = jnp.pad(x, ((0, NP - N), (0, 0)))
    zeros2 = jnp.zeros((NP, D), jnp.float32)
    zeros1 = jnp.zeros((NP,), jnp.float32)

    degp = _sc_degree(dst, zeros1)
    dpT = degp.T
    u = _tc_u(x_pad, W1, b1.reshape(1, D), Wg, dpT)
    p = _sc_scatter(u, src, dst, zeros2)
    out = _tc_final(p, dpT, bg.reshape(1, D), W2, b2.reshape(1, D))
    return out[:N]


# speedup vs baseline: 42.6080x; 1.0091x over previous
import functools

import jax
import jax.numpy as jnp
from jax import lax
from jax.experimental import pallas as pl
from jax.experimental.pallas import tpu as pltpu, tpu_sc as plsc

N = 10000
NP = 10240
D = 128
E = 320000
NC = 2
NS = 16
NW = NC * NS
K = 128
CB = 8
NB = 10
C = CB * NB
EP = NW * C * K
RPT = NP // NS

_mesh = plsc.VectorSubcoreMesh(core_axis_name="c", subcore_axis_name="s")


@functools.partial(
    pl.kernel,
    out_type=jax.ShapeDtypeStruct((NC, NP), jnp.float32),
    mesh=_mesh,
    scratch_types=[
        pltpu.VMEM((CB, K), jnp.int32),
        pltpu.VMEM((CB, K), jnp.int32),
        pltpu.VMEM((K,), jnp.float32),
        pltpu.VMEM_SHARED((NP,), jnp.float32),
        pltpu.SemaphoreType.DMA,
        pltpu.SemaphoreType.DMA,
        pltpu.SemaphoreType.DMA,
    ],
)
def _sc_degree(dst_hbm, zeros1_hbm, degp_hbm, iA, iB, ones_v, deg_sh,
               semA, semB, sem_i):
    c = lax.axis_index("c")
    s = lax.axis_index("s")
    wid = c * NS + s
    for t in range(K // 16):
        ones_v[pl.ds(t * 16, 16)] = jnp.ones((16,), jnp.float32)
    rows = pl.ds(pl.multiple_of(s * RPT, 8), RPT)
    pltpu.sync_copy(dst_hbm.at[wid, 0], iA)
    pltpu.sync_copy(zeros1_hbm.at[rows], deg_sh.at[rows])
    plsc.subcore_barrier()

    bufs, sems = (iA, iB), (semA, semB)
    for b in range(NB):
        me, sem_me = bufs[b % 2], sems[b % 2]
        nxt, sem_nxt = bufs[(b + 1) % 2], sems[(b + 1) % 2]
        if b > 0:
            pltpu.make_async_copy(dst_hbm.at[wid, b], me, sem_i).wait()
        for j in range(CB):
            pltpu.async_copy(ones_v, deg_sh.at[me.at[j]], sem_me, add=True)
        if b + 1 < NB:
            if b > 0:
                for j in range(CB):
                    pltpu.make_async_copy(ones_v, deg_sh.at[nxt.at[j]],
                                          sem_nxt).wait()
            pltpu.async_copy(dst_hbm.at[wid, b + 1], nxt, sem_i)
    for j in range(CB):
        pltpu.make_async_copy(ones_v, deg_sh.at[iA.at[j]], semA).wait()
    for j in range(CB):
        pltpu.make_async_copy(ones_v, deg_sh.at[iB.at[j]], semB).wait()
    plsc.subcore_barrier()

    @pl.when(s == 0)
    def _():
        pltpu.sync_copy(deg_sh, degp_hbm.at[c])


@functools.partial(
    pl.kernel,
    out_type=jax.ShapeDtypeStruct((NC, NP, D), jnp.float32),
    mesh=_mesh,
    scratch_types=[
        pltpu.VMEM((CB, K), jnp.int32),
        pltpu.VMEM((CB, K), jnp.int32),
        pltpu.VMEM((CB, K), jnp.int32),
        pltpu.VMEM((CB, K), jnp.int32),
        pltpu.VMEM((K, D), jnp.float32),
        pltpu.VMEM((K, D), jnp.float32),
        pltpu.VMEM_SHARED((NP, D), jnp.float32),
        pltpu.SemaphoreType.DMA,
        pltpu.SemaphoreType.DMA,
        pltpu.SemaphoreType.DMA,
    ],
)
def _sc_scatter(u_hbm, src_hbm, dst_hbm, zeros_hbm, p_hbm,
                isA, idA, isB, idB, r0, r1, agg_sh, sem0, sem1, sem_i):
    c = lax.axis_index("c")
    s = lax.axis_index("s")
    wid = c * NS + s
    rows = pl.ds(pl.multiple_of(s * RPT, 8), RPT)

    pltpu.sync_copy(src_hbm.at[wid, 0], isA)
    pltpu.sync_copy(dst_hbm.at[wid, 0], idA)
    pltpu.async_copy(u_hbm.at[isA.at[0]], r0, sem0)

    pltpu.sync_copy(zeros_hbm, agg_sh.at[rows])

    plsc.subcore_barrier()

    def do_block(b, is_v, id_v, nis, nid):
        @pl.when(b + 1 < NB)
        def _():
            pltpu.async_copy(src_hbm.at[wid, b + 1], nis, sem_i)
            pltpu.async_copy(dst_hbm.at[wid, b + 1], nid, sem_i)

        def body(j2, carry2):
            j = j2 * 2
            pltpu.async_copy(u_hbm.at[is_v.at[j + 1]], r1, sem1)
            pltpu.make_async_copy(u_hbm.at[is_v.at[j]], r0, sem0).wait()
            pltpu.sync_copy(r0, agg_sh.at[id_v.at[j]], add=True)

            @pl.when(j + 2 < CB)
            def _():
                pltpu.async_copy(u_hbm.at[is_v.at[j + 2]], r0, sem0)

            @pl.when(jnp.logical_and(j2 == CB // 2 - 1, b + 1 < NB))
            def _():
                pltpu.make_async_copy(src_hbm.at[wid, b + 1], nis, sem_i).wait()
                pltpu.make_async_copy(dst_hbm.at[wid, b + 1], nid, sem_i).wait()
                pltpu.async_copy(u_hbm.at[nis.at[0]], r0, sem0)

            pltpu.make_async_copy(u_hbm.at[is_v.at[j + 1]], r1, sem1).wait()
            pltpu.sync_copy(r1, agg_sh.at[id_v.at[j + 1]], add=True)
            return carry2

        lax.fori_loop(0, CB // 2, body, 0)

    def blkpair(bb, carry):
        b = bb * 2
        do_block(b, isA, idA, isB, idB)
        do_block(b + 1, isB, idB, isA, idA)
        return carry

    lax.fori_loop(0, NB // 2, blkpair, 0)
    plsc.subcore_barrier()
    pltpu.sync_copy(agg_sh.at[rows], p_hbm.at[c, rows])


def _u_body(x_ref, w1_ref, b1_ref, wg_ref, dp_ref, u_ref):
    h = lax.dot_general(x_ref[...], w1_ref[...], (((1,), (1,)), ((), ())),
                        preferred_element_type=jnp.float32)
    h = jnp.maximum(h + b1_ref[...], 0.0)
    h2 = lax.dot_general(h, wg_ref[...], (((1,), (1,)), ((), ())),
                         preferred_element_type=jnp.float32)
    deg = 1.0 + dp_ref[:, 0:1] + dp_ref[:, 1:2]
    u_ref[...] = h2 * lax.rsqrt(deg)


def _f_body(p_ref, u_ref, dp_ref, bg_ref, w2_ref, b2_ref, o_ref):
    ssum = p_ref[0] + p_ref[1] + u_ref[...]
    deg = 1.0 + dp_ref[:, 0:1] + dp_ref[:, 1:2]
    gcn = ssum * lax.rsqrt(deg) + bg_ref[...]
    o_ref[...] = lax.dot_general(gcn, w2_ref[...], (((1,), (1,)), ((), ())),
                                 preferred_element_type=jnp.float32) + b2_ref[...]


_R = 1000


def _tc_u(x, W1, b1r, Wg, dpT):
    return pl.pallas_call(
        _u_body,
        grid=(N // _R,),
        in_specs=[
            pl.BlockSpec((_R, D), lambda i: (i, 0)),
            pl.BlockSpec((D, D), lambda i: (0, 0)),
            pl.BlockSpec((1, D), lambda i: (0, 0)),
            pl.BlockSpec((D, D), lambda i: (0, 0)),
            pl.BlockSpec((_R, 2), lambda i: (i, 0)),
        ],
        out_specs=pl.BlockSpec((_R, D), lambda i: (i, 0)),
        out_shape=jax.ShapeDtypeStruct((N, D), jnp.float32),
    )(x, W1, b1r, Wg, dpT)


def _tc_final(p, u, dpT, bgr, W2, b2r):
    return pl.pallas_call(
        _f_body,
        grid=(N // _R,),
        in_specs=[
            pl.BlockSpec((NC, _R, D), lambda i: (0, i, 0)),
            pl.BlockSpec((_R, D), lambda i: (i, 0)),
            pl.BlockSpec((_R, 2), lambda i: (i, 0)),
            pl.BlockSpec((1, D), lambda i: (0, 0)),
            pl.BlockSpec((D, D), lambda i: (0, 0)),
            pl.BlockSpec((1, D), lambda i: (0, 0)),
        ],
        out_specs=pl.BlockSpec((_R, D), lambda i: (i, 0)),
        out_shape=jax.ShapeDtypeStruct((N, D), jnp.float32),
    )(p, u, dpT, bgr, W2, b2r)


def kernel(x, edge_index, W1, b1, Wg, bg, W2, b2):
    pad = jnp.arange(EP - E, dtype=jnp.int32)
    src = jnp.concatenate([edge_index[0], pad % N]).reshape(NW, NB, CB, K)
    dst = jnp.concatenate([edge_index[1], N + pad % (NP - N)]
                          ).reshape(NW, NB, CB, K)
    zeros3 = jnp.zeros((RPT, D), jnp.float32)
    zeros1 = jnp.zeros((NP,), jnp.float32)

    degp = _sc_degree(dst, zeros1)
    dpT = degp.T[:N]
    u = _tc_u(x, W1, b1.reshape(1, D), Wg, dpT)
    p = _sc_scatter(u, src, dst, zeros3)
    return _tc_final(p, u, dpT, bg.reshape(1, D), W2, b2.reshape(1, D))
